# Initial kernel scaffold; baseline (speedup 1.0000x reference)
import jax
import jax.numpy as jnp
from jax.experimental import pallas as pl


def kernel(tokens, c0, h0, table, wi, bi, wh, bh):
    raise NotImplementedError("write your pallas kernel here")


# speedup vs baseline: 1.4681x; 1.4681x over previous
import jax
import jax.numpy as jnp
from jax.experimental import pallas as pl
from jax.experimental.pallas import tpu as pltpu


def _lstm_step_kernel(x_ref,
                      w_ref,
                      b_ref,
                      c0_ref,
                      h0_ref,
                      cy_ref,
                      hy_ref,
                      c_st, h_st):
    t = pl.program_id(1)
    H = c0_ref.shape[1]

    @pl.when(t == 0)
    def _():
        c_st[...] = c0_ref[...]
        h_st[...] = h0_ref[...]

    xh = jnp.concatenate([x_ref[0], h_st[...]], axis=-1)
    gates = jnp.dot(xh, w_ref[...],
                    preferred_element_type=jnp.float32) + b_ref[...]

    ingate     = jax.nn.sigmoid(gates[:, 0 * H:1 * H])
    forgetgate = jax.nn.sigmoid(gates[:, 1 * H:2 * H])
    cellgate   = jnp.tanh(gates[:, 2 * H:3 * H])
    outgate    = jax.nn.sigmoid(gates[:, 3 * H:4 * H])

    cy = forgetgate * c_st[...] + ingate * cellgate
    hy = outgate * jnp.tanh(cy)

    c_st[...] = cy
    h_st[...] = hy
    cy_ref[0] = cy
    hy_ref[0] = hy


def kernel(tokens, c0, h0, table, wi, bi, wh, bh):
    T, B = tokens.shape
    V, H = table.shape
    Bt = B // 2 if B % 2 == 0 else B

    x_emb = jnp.take(table, tokens, axis=0)
    w = jnp.concatenate([wi, wh], axis=0)
    b = bi + bh

    cy_seq, hy_seq = pl.pallas_call(
        _lstm_step_kernel,
        out_shape=(jax.ShapeDtypeStruct((T, B, H), jnp.float32),
                   jax.ShapeDtypeStruct((T, B, H), jnp.float32)),
        grid=(B // Bt, T),
        in_specs=[
            pl.BlockSpec((1, Bt, H),    lambda bidx, t: (t, bidx, 0)),
            pl.BlockSpec((2 * H, 4 * H), lambda bidx, t: (0, 0)),
            pl.BlockSpec((1, 4 * H),    lambda bidx, t: (0, 0)),
            pl.BlockSpec((Bt, H),       lambda bidx, t: (bidx, 0)),
            pl.BlockSpec((Bt, H),       lambda bidx, t: (bidx, 0)),
        ],
        out_specs=(pl.BlockSpec((1, Bt, H), lambda bidx, t: (t, bidx, 0)),
                   pl.BlockSpec((1, Bt, H), lambda bidx, t: (t, bidx, 0))),
        scratch_shapes=[
            pltpu.VMEM((Bt, H), jnp.float32),
            pltpu.VMEM((Bt, H), jnp.float32),
        ],
        compiler_params=pltpu.CompilerParams(
            dimension_semantics=("parallel", "arbitrary"),
            vmem_limit_bytes=48 * 1024 * 1024,
        ),
    )(x_emb, w, b, c0, h0)

    return cy_seq, hy_seq
